# Initial kernel scaffold; baseline (speedup 1.0000x reference)
import jax
import jax.numpy as jnp
from jax.experimental import pallas as pl


def kernel(hidden_states, gate_w, w_gate, w_up, w_down):
    raise NotImplementedError("write your pallas kernel here")


# speedup vs baseline: 1.0264x; 1.0264x over previous
import functools

import jax
import jax.numpy as jnp
from jax import lax
from jax.experimental import pallas as pl
from jax.experimental.pallas import tpu as pltpu
from jax.experimental.pallas import tpu_sc as plsc

_E, _K, _D, _F, _S = 8, 2, 2048, 1024, 2048
_M = 128
_NB = (2 * _S) // _M + _E
_MP = _NB * _M
_NC, _NS, _NL = 2, 16, 16
_NW = _NC * _NS
_PT = _MP // _NW
_GC = 40
_TPT = _S // _NW
_CT = 8


def _router_body(x_ref, gw_ref, pos_ref, went_ref, gblk_ref, nval_ref):
    x = x_ref[...]
    logits = lax.dot_general(x, gw_ref[...], (((1,), (1,)), ((), ())),
                             preferred_element_type=jnp.float32)
    m = jnp.max(logits, axis=1, keepdims=True)
    ex = jnp.exp(logits - m)
    probs = ex / jnp.sum(ex, axis=1, keepdims=True)
    lane = lax.broadcasted_iota(jnp.int32, (_S, _E), 1)
    m0 = jnp.max(probs, axis=1, keepdims=True)
    e0 = jnp.min(jnp.where(probs == m0, lane, _E), axis=1, keepdims=True)
    probs2 = jnp.where(lane == e0, -1.0, probs)
    m1 = jnp.max(probs2, axis=1, keepdims=True)
    e1 = jnp.min(jnp.where(probs2 == m1, lane, _E), axis=1, keepdims=True)
    e_all = jnp.concatenate([e0, e1], axis=0)
    w_all = jnp.concatenate([m0, m1], axis=0)
    lane2 = lax.broadcasted_iota(jnp.int32, (2 * _S, _E), 1)
    onehot = (lane2 == e_all).astype(jnp.float32)
    ch = 512
    r_i = lax.broadcasted_iota(jnp.int32, (ch, ch), 0)
    c_i = lax.broadcasted_iota(jnp.int32, (ch, ch), 1)
    lstrict = (c_i < r_i).astype(jnp.float32)
    carry = jnp.zeros((1, _E), jnp.float32)
    ranks = []
    for c in range((2 * _S) // ch):
        oc = onehot[c * ch:(c + 1) * ch]
        within = lax.dot_general(lstrict, oc, (((1,), (0,)), ((), ())),
                                 preferred_element_type=jnp.float32)
        ranks.append(within + carry)
        carry = carry + jnp.sum(oc, axis=0, keepdims=True)
    rank = jnp.concatenate(ranks, axis=0)
    rank_e = jnp.sum(rank * onehot, axis=1, keepdims=True)
    counts = carry.astype(jnp.int32)
    padded = ((counts + (_M - 1)) // _M) * _M
    inc = padded
    for sh in (1, 2, 4):
        z = jnp.zeros((1, sh), jnp.int32)
        inc = inc + jnp.concatenate([z, inc[:, :-sh]], axis=1)
    excl = inc - padded
    off_e = jnp.sum(jnp.where(lane2 == e_all,
                              jnp.broadcast_to(excl, (2 * _S, _E)), 0),
                    axis=1, keepdims=True)
    pos_ref[...] = off_e + rank_e.astype(jnp.int32)
    went_ref[...] = w_all
    bstart = lax.broadcasted_iota(jnp.int32, (_NB, _E), 0) * _M
    g = jnp.sum((bstart >= jnp.broadcast_to(inc, (_NB, _E))).astype(jnp.int32),
                axis=1, keepdims=True)
    gblk_ref[...] = jnp.minimum(g, _E - 1)
    nval_ref[...] = inc[:, _E - 1:] // _M


_router_call = pl.pallas_call(
    _router_body,
    out_shape=(
        jax.ShapeDtypeStruct((2 * _S, 1), jnp.int32),
        jax.ShapeDtypeStruct((2 * _S, 1), jnp.float32),
        jax.ShapeDtypeStruct((_NB, 1), jnp.int32),
        jax.ShapeDtypeStruct((1, 1), jnp.int32),
    ),
)


def _expert_body(gblk_ref, nval_ref, xs_ref, wg_ref, wu_ref, wd_ref, ws_ref,
                 out_ref):
    i = pl.program_id(0)

    @pl.when(i < nval_ref[0])
    def _():
        xb = xs_ref[...]
        g = lax.dot_general(xb, wg_ref[0], (((1,), (1,)), ((), ())),
                            preferred_element_type=jnp.float32)
        u = lax.dot_general(xb, wu_ref[0], (((1,), (1,)), ((), ())),
                            preferred_element_type=jnp.float32)
        h = g * u / (1.0 + jnp.exp(-g))
        y = lax.dot_general(h, wd_ref[0], (((1,), (1,)), ((), ())),
                            preferred_element_type=jnp.float32)
        out_ref[...] = y * ws_ref[...]

    @pl.when(i >= nval_ref[0])
    def _():
        out_ref[...] = jnp.zeros_like(out_ref)


_expert_call = pl.pallas_call(
    _expert_body,
    grid_spec=pltpu.PrefetchScalarGridSpec(
        num_scalar_prefetch=2,
        grid=(_NB,),
        in_specs=[
            pl.BlockSpec((_M, _D), lambda i, g, n: (i, 0)),
            pl.BlockSpec((1, _F, _D), lambda i, g, n: (g[i], 0, 0)),
            pl.BlockSpec((1, _F, _D), lambda i, g, n: (g[i], 0, 0)),
            pl.BlockSpec((1, _D, _F), lambda i, g, n: (g[i], 0, 0)),
            pl.BlockSpec((_M, 1), lambda i, g, n: (i, 0)),
        ],
        out_specs=pl.BlockSpec((_M, _D), lambda i, g, n: (i, 0)),
    ),
    out_shape=jax.ShapeDtypeStruct((_MP, _D), jnp.float32),
)


def _dispatch_body(x_hbm, pos_hbm, w_hbm, xs_hbm, ws_hbm,
                   pos_v, w_v, tok_v, ws_v, rowbuf, sem):
    wid = lax.axis_index("s") * _NC + lax.axis_index("c")
    pltpu.sync_copy(pos_hbm, pos_v)
    pltpu.sync_copy(w_hbm, w_v)

    def _init(i, c):
        sl = pl.ds(pl.multiple_of(i * _NL, _NL), _NL)
        tok_v[sl] = jnp.zeros((_NL,), jnp.int32)
        ws_v[sl] = jnp.zeros((_NL,), jnp.float32)
        return c

    lax.fori_loop(0, _MP // _NL, _init, 0)

    def _scat(i, c):
        sl = pl.ds(pl.multiple_of(i * _NL, _NL), _NL)
        idx = pos_v[sl]
        j = i * _NL + lax.iota(jnp.int32, _NL)
        plsc.store_scatter(tok_v, [idx], jnp.bitwise_and(j, _S - 1))
        plsc.store_scatter(ws_v, [idx], w_v[sl])
        return c

    lax.fori_loop(0, (2 * _S) // _NL, _scat, 0)

    base = pl.multiple_of(wid * _PT, 8)
    pltpu.sync_copy(ws_v.at[pl.ds(base, _PT)], ws_hbm.at[pl.ds(base, _PT)])
    for chnk in range(_PT // _GC):
        st = pl.multiple_of(wid * _PT + chnk * _GC, 8)
        pltpu.async_copy(x_hbm.at[tok_v.at[pl.ds(st, _GC)]], rowbuf, sem).wait()
        pltpu.sync_copy(rowbuf, xs_hbm.at[pl.ds(st, _GC)])


def _combine_body(ys_hbm, pos_hbm, out_hbm, pos_v, buf0, buf1, sem0, sem1):
    wid = lax.axis_index("s") * _NC + lax.axis_index("c")
    pltpu.sync_copy(pos_hbm, pos_v)

    def _chunk(ci, c):
        tb = pl.multiple_of(wid * _TPT + ci * _CT, 8)
        cp0 = pltpu.async_copy(ys_hbm.at[pos_v.at[pl.ds(tb, _CT)]], buf0, sem0)
        cp1 = pltpu.async_copy(
            ys_hbm.at[pos_v.at[pl.ds(_S + tb, _CT)]], buf1, sem1)
        cp0.wait()
        cp1.wait()

        def _addrow(r, c2):
            for cc in range(_D // _NL):
                sl = pl.ds(cc * _NL, _NL)
                buf0[r, sl] = buf0[r, sl] + buf1[r, sl]
            return c2

        lax.fori_loop(0, _CT, _addrow, 0)
        pltpu.sync_copy(buf0, out_hbm.at[pl.ds(tb, _CT)])
        return c

    lax.fori_loop(0, _TPT // _CT, _chunk, 0)


@functools.cache
def _sc_calls():
    mesh = plsc.VectorSubcoreMesh(core_axis_name="c", subcore_axis_name="s")
    dispatch = pl.kernel(
        _dispatch_body,
        mesh=mesh,
        compiler_params=pltpu.CompilerParams(needs_layout_passes=False),
        out_type=(
            jax.ShapeDtypeStruct((_MP, _D), jnp.float32),
            jax.ShapeDtypeStruct((_MP,), jnp.float32),
        ),
        scratch_types=[
            pltpu.VMEM((2 * _S,), jnp.int32),
            pltpu.VMEM((2 * _S,), jnp.float32),
            pltpu.VMEM((_MP,), jnp.int32),
            pltpu.VMEM((_MP,), jnp.float32),
            pltpu.VMEM((_GC, _D), jnp.float32),
            pltpu.SemaphoreType.DMA,
        ],
    )
    combine = pl.kernel(
        _combine_body,
        mesh=mesh,
        compiler_params=pltpu.CompilerParams(needs_layout_passes=False),
        out_type=jax.ShapeDtypeStruct((_S, _D), jnp.float32),
        scratch_types=[
            pltpu.VMEM((2 * _S,), jnp.int32),
            pltpu.VMEM((_CT, _D), jnp.float32),
            pltpu.VMEM((_CT, _D), jnp.float32),
            pltpu.SemaphoreType.DMA,
            pltpu.SemaphoreType.DMA,
        ],
    )
    return dispatch, combine


def kernel(hidden_states, gate_w, w_gate, w_up, w_down):
    b, s, d = hidden_states.shape
    x = hidden_states.reshape(s, d)
    pos2, went2, gblk2, nval2 = _router_call(x, gate_w)
    pos = pos2.reshape(2 * s)
    went = went2.reshape(2 * s)
    gblk = gblk2.reshape(_NB)
    nval = nval2.reshape(1)
    dispatch, combine = _sc_calls()
    xs, ws = dispatch(x, pos, went)
    ys = _expert_call(gblk, nval, xs, w_gate, w_up, w_down, ws.reshape(_MP, 1))
    out = combine(ys, pos)
    return out.reshape(b, s, d)
